# Initial kernel scaffold; baseline (speedup 1.0000x reference)
import jax
import jax.numpy as jnp
from jax.experimental import pallas as pl


def kernel(x, edge_index, target_node, current_prereq_mask, node_emb, Wp, bp, Wg, bg, Wd, bd, Wa, ba, prereq_weight):
    raise NotImplementedError("write your pallas kernel here")


# speedup vs baseline: 33.6695x; 33.6695x over previous
import functools

import jax
import jax.numpy as jnp
from jax import lax
from jax.experimental import pallas as pl
from jax.experimental.pallas import tpu as pltpu
from jax.experimental.pallas import tpu_sc as plsc

N = 50000
E = 800000
H = 64

NP = 50176
EPAD = 819200
EROWS = EPAD // 128

NC = 2
NS = 16


G_DEG = EROWS // (NC * NS)
ZR = NP // NS


def _deg_body(dst2d, out0, out1, idxb, onesb, zbuf, hist):
    cid = lax.axis_index("c")
    sid = lax.axis_index("s")
    wid = sid * NC + cid

    ones16 = jnp.ones((16,), jnp.float32)
    for k in range(128 // 16):
        onesb[pl.ds(k * 16, 16)] = ones16

    zeros16 = jnp.zeros((16,), jnp.float32)

    def _z(i, _):
        zbuf[pl.ds(i * 16, 16)] = zeros16
        return 0

    lax.fori_loop(0, ZR // 16, _z, 0)
    pltpu.sync_copy(zbuf, hist.at[pl.ds(sid * ZR, ZR)])

    pltpu.sync_copy(dst2d.at[pl.ds(wid * G_DEG, G_DEG)], idxb)
    plsc.subcore_barrier()

    def _scatter(j, _):
        pltpu.sync_copy(onesb, hist.at[idxb.at[j]], add=True)
        return 0

    lax.fori_loop(0, G_DEG, _scatter, 0)
    plsc.subcore_barrier()

    pltpu.sync_copy(hist.at[pl.ds(sid * ZR, ZR)], zbuf)

    @pl.when(cid == 0)
    def _():
        pltpu.sync_copy(zbuf, out0.at[pl.ds(sid * ZR, ZR)])

    @pl.when(cid == 1)
    def _():
        pltpu.sync_copy(zbuf, out1.at[pl.ds(sid * ZR, ZR)])


def _degree_histogram(dst2d):
    mesh = plsc.VectorSubcoreMesh(core_axis_name="c", subcore_axis_name="s")
    return pl.kernel(
        _deg_body,
        out_type=(jax.ShapeDtypeStruct((NP,), jnp.float32),
                  jax.ShapeDtypeStruct((NP,), jnp.float32)),
        mesh=mesh,
        scratch_types=[
            pltpu.VMEM((G_DEG, 128), jnp.int32),
            pltpu.VMEM((128,), jnp.float32),
            pltpu.VMEM((ZR,), jnp.float32),
            pltpu.VMEM_SHARED((NP,), jnp.float32),
        ],
    )(dst2d)


K_CH = EROWS // NS
G_STG = 40
N_STG = K_CH // G_STG
ZCH = 112
WCH = 64


def _gs_stage(u_ref, acc, sidx, didx, rows, sem0, sem1):
    sems = (sem0, sem1)

    def _start(j, b):
        return pltpu.async_copy(u_ref.at[sidx.at[j]], rows.at[b], sems[b])

    _start(0, 0)

    def _body(i, _):
        for b in (0, 1):
            j = i * 2 + b
            nb = 1 - b

            @pl.when(j + 1 < G_STG)
            def _():
                _start(j + 1, nb)

            pltpu.make_async_copy(u_ref.at[sidx.at[j]], rows.at[b], sems[b]).wait()
            pltpu.sync_copy(rows.at[b], acc.at[didx.at[j]], add=True)
        return 0

    lax.fori_loop(0, G_STG // 2, _body, 0)


def _scatter_body(u0, u1, src2d, dst2d, out0, out1, sidx, didx, rows,
                  acc, sem0, sem1):
    cid = lax.axis_index("c")
    sid = lax.axis_index("s")

    zeros16 = jnp.zeros((16,), jnp.float32)

    def _z(i, _):
        rows[0, i, pl.ds(0, 16)] = zeros16
        rows[0, i, pl.ds(16, 16)] = zeros16
        return 0

    lax.fori_loop(0, 128, _z, 0)
    zsrc = rows.at[0].at[pl.ds(0, ZCH)]

    def _zdma(k, _):
        pltpu.sync_copy(zsrc, acc.at[pl.ds(sid * ZR + k * ZCH, ZCH)])
        return 0

    lax.fori_loop(0, ZR // ZCH, _zdma, 0)
    plsc.subcore_barrier()

    def _stage(s, _):
        pltpu.sync_copy(src2d.at[pl.ds(sid * K_CH + s * G_STG, G_STG)], sidx)
        pltpu.sync_copy(dst2d.at[pl.ds(sid * K_CH + s * G_STG, G_STG)], didx)

        @pl.when(cid == 0)
        def _():
            _gs_stage(u0, acc, sidx, didx, rows, sem0, sem1)

        @pl.when(cid == 1)
        def _():
            _gs_stage(u1, acc, sidx, didx, rows, sem0, sem1)

        return 0

    lax.fori_loop(0, N_STG, _stage, 0)
    plsc.subcore_barrier()

    wbuf = rows.at[1].at[pl.ds(0, WCH)]

    def _wout(t, _):
        r0 = sid * ZR + t * WCH
        pltpu.sync_copy(acc.at[pl.ds(r0, WCH)], wbuf)

        @pl.when(cid == 0)
        def _():
            pltpu.sync_copy(wbuf, out0.at[pl.ds(r0, WCH)])

        @pl.when(cid == 1)
        def _():
            pltpu.sync_copy(wbuf, out1.at[pl.ds(r0, WCH)])

        return 0

    lax.fori_loop(0, ZR // WCH, _wout, 0)


def _edge_scatter(u0, u1, src2d, dst2d):
    mesh = plsc.VectorSubcoreMesh(core_axis_name="c", subcore_axis_name="s")
    return pl.kernel(
        _scatter_body,
        out_type=(jax.ShapeDtypeStruct((NP, 32), jnp.float32),
                  jax.ShapeDtypeStruct((NP, 32), jnp.float32)),
        mesh=mesh,
        scratch_types=[
            pltpu.VMEM((G_STG, 128), jnp.int32),
            pltpu.VMEM((G_STG, 128), jnp.int32),
            pltpu.VMEM((2, 128, 32), jnp.float32),
            pltpu.VMEM_SHARED((NP, 32), jnp.float32),
            pltpu.SemaphoreType.DMA,
            pltpu.SemaphoreType.DMA,
        ],
        compiler_params=pltpu.CompilerParams(use_tc_tiling_on_sc=False),
    )(u0, u1, src2d, dst2d)


BR = 1568
GRID = NP // BR


def _dense_body(x_ref, emb_ref, d0_ref, d1_ref, wpt_ref, bp_ref, wgt_ref,
                u0_ref, u1_ref, dis_ref):
    deg = d0_ref[...] + d1_ref[...] + 1.0
    dis = lax.rsqrt(deg)
    h = emb_ref[...] + jnp.dot(x_ref[...], wpt_ref[...],
                               preferred_element_type=jnp.float32) + bp_ref[...]
    hW = jnp.dot(h, wgt_ref[...], preferred_element_type=jnp.float32)
    u = hW * dis
    u0_ref[...] = u[:, :32]
    u1_ref[...] = u[:, 32:]
    dis_ref[...] = dis


def _dense_prepass(x_pad, emb_pad, deg0, deg1, WpT, bp2, WgT):
    bs = lambda cols: pl.BlockSpec((BR, cols), lambda i: (i, 0))
    full = lambda r, c: pl.BlockSpec((r, c), lambda i: (0, 0))
    return pl.pallas_call(
        _dense_body,
        grid=(GRID,),
        in_specs=[bs(2), bs(64), bs(1), bs(1), full(2, 64), full(1, 64),
                  full(64, 64)],
        out_specs=[bs(32), bs(32), bs(1)],
        out_shape=[jax.ShapeDtypeStruct((NP, 32), jnp.float32),
                   jax.ShapeDtypeStruct((NP, 32), jnp.float32),
                   jax.ShapeDtypeStruct((NP, 1), jnp.float32)],
    )(x_pad, emb_pad, deg0.reshape(NP, 1), deg1.reshape(NP, 1), WpT, bp2, WgT)


def _final_body(a0_ref, a1_ref, u0_ref, u1_ref, dis_ref, x_ref, m_ref,
                bg_ref, wd_ref, wa_ref, sc_ref, tgt_ref,
                prob_ref, gap_ref, vacc, sacc):
    i = pl.program_id(0)

    @pl.when(i == 0)
    def _():
        vacc[...] = jnp.zeros_like(vacc)
        sacc[0] = 0.0
        sacc[1] = 0.0

    acc = jnp.concatenate([a0_ref[...], a1_ref[...]], axis=1)
    u = jnp.concatenate([u0_ref[...], u1_ref[...]], axis=1)
    z = jax.nn.relu(dis_ref[...] * (acc + u) + bg_ref[...])

    rows = i * BR + lax.broadcasted_iota(jnp.int32, (BR, 1), 0)
    tsel = (rows == tgt_ref[0]).astype(jnp.float32)
    score = x_ref[:, 1:2]

    vacc[0:1, :] = vacc[0:1, :] + jnp.sum(z * tsel, axis=0, keepdims=True)
    vacc[1:2, :] = vacc[1:2, :] + jnp.sum(z * score, axis=0, keepdims=True)
    sacc[0] = sacc[0] + jnp.sum(x_ref[:, 0])
    sacc[1] = sacc[1] + jnp.sum(jax.nn.softplus(z) * m_ref[...])

    @pl.when(i == pl.num_programs(0) - 1)
    def _():
        difficulty = jnp.sum(vacc[0:1, :] * wd_ref[...]) + sc_ref[0]
        n_learned = jnp.maximum(sacc[0], 1.0)
        ability = jnp.sum((vacc[1:2, :] / n_learned) * wa_ref[...]) + sc_ref[1]
        prereq = jnp.abs(sc_ref[2]) * (sacc[1] / 64.0)
        gap = ability - difficulty + prereq
        ones11 = jnp.ones((1, 1), jnp.float32)
        gap_ref[...] = gap * ones11
        prob_ref[...] = jax.nn.sigmoid(gap) * ones11


def _final_pass(a0, a1, u0, u1, dis, x_pad, mask_pad, bg2, Wd, Wa, scalars,
                tgt):
    bs = lambda cols: pl.BlockSpec((BR, cols), lambda i: (i, 0))
    full = lambda r, c: pl.BlockSpec((r, c), lambda i: (0, 0))
    return pl.pallas_call(
        _final_body,
        grid=(GRID,),
        in_specs=[bs(32), bs(32), bs(32), bs(32), bs(1), bs(2), bs(1),
                  full(1, 64), full(1, 64), full(1, 64),
                  pl.BlockSpec(memory_space=pltpu.SMEM),
                  pl.BlockSpec(memory_space=pltpu.SMEM)],
        out_specs=[full(1, 1), full(1, 1)],
        out_shape=[jax.ShapeDtypeStruct((1, 1), jnp.float32),
                   jax.ShapeDtypeStruct((1, 1), jnp.float32)],
        scratch_shapes=[pltpu.VMEM((2, 64), jnp.float32),
                        pltpu.SMEM((2,), jnp.float32)],
    )(a0, a1, u0, u1, dis, x_pad, mask_pad, bg2, Wd, Wa, scalars, tgt)


def kernel(x, edge_index, target_node, current_prereq_mask, node_emb,
           Wp, bp, Wg, bg, Wd, bd, Wa, ba, prereq_weight):
    src = edge_index[0]
    dst = edge_index[1]
    pad = EPAD - E
    trash = (N + (jnp.arange(pad, dtype=jnp.int32) % (NP - N))).astype(jnp.int32)
    src2d = jnp.concatenate([src, trash]).reshape(EROWS, 128)
    dst2d = jnp.concatenate([dst, trash]).reshape(EROWS, 128)

    pad2 = lambda a: jnp.pad(a, ((0, NP - N), (0, 0)))
    x_pad = pad2(x)
    emb_pad = pad2(node_emb)
    mask_pad = pad2(current_prereq_mask[:, None])

    deg0, deg1 = _degree_histogram(dst2d)

    u0, u1, dis = _dense_prepass(x_pad, emb_pad, deg0, deg1, Wp.T,
                                 bp.reshape(1, 64), Wg.T)

    a0, a1 = _edge_scatter(u0, u1, src2d, dst2d)

    scalars = jnp.stack([bd[0], ba[0], prereq_weight[0]])
    tgt = jnp.asarray(target_node, jnp.int32).reshape(1)
    prob2, gap2 = _final_pass(a0, a1, u0, u1, dis, x_pad, mask_pad,
                              bg.reshape(1, 64), Wd, Wa, scalars, tgt)
    return (prob2[0, 0], gap2[0, 0])
